# baseline (speedup 1.0000x reference)
import functools

import jax
import jax.numpy as jnp
from jax.experimental import pallas as pl
from jax.experimental.pallas import tpu as pltpu

D_MODEL_ = 768
D_FF_ = 3072
N_EXPERTS_ = 8
N_HEADS_ = 12
S_ = 2048
DH_ = D_MODEL_ // N_HEADS_

TB = 256
QB = 512
FC = 1536
CHUNKS_PER_EXPERT = D_FF_ // FC
N_CHUNKS = N_EXPERTS_ * CHUNKS_PER_EXPERT


def _attn_kernel(x_ref, wq_ref, wk_ref, wv_ref, bq_ref, bk_ref, bv_ref,
                 wo_ref, bo_ref, g_ref, b_ref,
                 attn_ref, x1_ref, xb_scr, k_scr, v_scr, t_scr, *, scale):
    qb = pl.program_id(0)
    hh = pl.program_id(1)

    @pl.when(jnp.logical_and(hh == 0, qb == 0))
    def _xcast():
        xb_scr[...] = x_ref[...].astype(jnp.bfloat16)

    @pl.when(qb == 0)
    def _kv():
        xb = xb_scr[...]
        k = jnp.dot(xb, wk_ref[0], preferred_element_type=jnp.float32) + bk_ref[0]
        v = jnp.dot(xb, wv_ref[0], preferred_element_type=jnp.float32) + bv_ref[0]
        k_scr[hh] = k.astype(jnp.bfloat16)
        v_scr[hh] = v.astype(jnp.bfloat16)

    @pl.when(hh == 0)
    def _resid():
        t_scr[...] = x_ref[pl.ds(qb * QB, QB), :] + bo_ref[...]

    xq = xb_scr[pl.ds(qb * QB, QB), :]
    q = jnp.dot(xq, wq_ref[0], preferred_element_type=jnp.float32) + bq_ref[0]
    s = jax.lax.dot_general(q.astype(jnp.bfloat16), k_scr[hh],
                            (((1,), (1,)), ((), ())),
                            preferred_element_type=jnp.float32) * scale
    m = jnp.max(s, axis=-1, keepdims=True)
    p = jnp.exp(s - m)
    a = p * (1.0 / jnp.sum(p, axis=-1, keepdims=True))
    attn_ref[...] = a[None]
    ctx = jnp.dot(a.astype(jnp.bfloat16), v_scr[hh],
                  preferred_element_type=jnp.float32)
    t_scr[...] += jnp.dot(ctx.astype(jnp.bfloat16), wo_ref[0],
                          preferred_element_type=jnp.float32)

    @pl.when(hh == N_HEADS_ - 1)
    def _ln():
        t = t_scr[...]
        mu = jnp.mean(t, axis=-1, keepdims=True)
        d = t - mu
        var = jnp.mean(d * d, axis=-1, keepdims=True)
        x1_ref[...] = d * jax.lax.rsqrt(var + 1e-5) * g_ref[...] + b_ref[...]


def _moe_kernel(x1_ref, wg_ref, w1_ref, b1_ref, w2_ref, b2_ref, g2_ref, be2_ref,
                out_ref, aux_ref, probs_scr, acc_scr):
    kk = pl.program_id(0)

    @pl.when(kk == 0)
    def _init():
        logits = jnp.dot(x1_ref[...], wg_ref[...], preferred_element_type=jnp.float32)
        lm = jnp.max(logits, axis=-1, keepdims=True)
        pe = jnp.exp(logits - lm)
        p = pe / jnp.sum(pe, axis=-1, keepdims=True)
        probs_scr[...] = p
        imp = jnp.mean(p, axis=0)
        aux_ref[...] = (N_EXPERTS_ * jnp.sum(imp * imp)).reshape(1, 1)
        acc_scr[...] = jnp.zeros_like(acc_scr)

    e = kk // CHUNKS_PER_EXPERT
    h = jnp.dot(x1_ref[...], w1_ref[0], preferred_element_type=jnp.float32) + b1_ref[0]
    h = jnp.maximum(h, 0.0)
    onehot = (jax.lax.broadcasted_iota(jnp.int32, (N_EXPERTS_, 1), 0) == e
              ).astype(jnp.float32)
    pcol = jnp.dot(probs_scr[...], onehot, preferred_element_type=jnp.float32)
    acc_scr[...] += pcol * jnp.dot(h, w2_ref[0], preferred_element_type=jnp.float32)

    @pl.when(kk == N_CHUNKS - 1)
    def _fin():
        pb2 = jnp.dot(probs_scr[...], b2_ref[...], preferred_element_type=jnp.float32)
        t = x1_ref[...] + acc_scr[...] + pb2
        mu = jnp.mean(t, axis=-1, keepdims=True)
        d = t - mu
        var = jnp.mean(d * d, axis=-1, keepdims=True)
        out_ref[...] = d * jax.lax.rsqrt(var + 1e-5) * g2_ref[...] + be2_ref[...]


def kernel(x, Wq, bq, Wk, bk, Wv, bv, Wo, bo, g1, be1, g2, be2, Wg, W1, b1, W2, b2):
    Bx, S, D = x.shape
    H, DH = N_HEADS_, DH_
    x2d = x.reshape(S, D)

    Wqh = Wq.reshape(D, H, DH).transpose(1, 0, 2).astype(jnp.bfloat16)
    Wkh = Wk.reshape(D, H, DH).transpose(1, 0, 2).astype(jnp.bfloat16)
    Wvh = Wv.reshape(D, H, DH).transpose(1, 0, 2).astype(jnp.bfloat16)
    Woh = Wo.reshape(H, DH, D).astype(jnp.bfloat16)
    bqh = bq.reshape(H, 1, DH)
    bkh = bk.reshape(H, 1, DH)
    bvh = bv.reshape(H, 1, DH)

    nq = S // QB
    attn, x1 = pl.pallas_call(
        functools.partial(_attn_kernel, scale=1.0 / (DH ** 0.5)),
        grid=(nq, H),
        in_specs=[
            pl.BlockSpec((S, D), lambda i, h: (0, 0)),
            pl.BlockSpec((1, D, DH), lambda i, h: (h, 0, 0)),
            pl.BlockSpec((1, D, DH), lambda i, h: (h, 0, 0)),
            pl.BlockSpec((1, D, DH), lambda i, h: (h, 0, 0)),
            pl.BlockSpec((1, 1, DH), lambda i, h: (h, 0, 0)),
            pl.BlockSpec((1, 1, DH), lambda i, h: (h, 0, 0)),
            pl.BlockSpec((1, 1, DH), lambda i, h: (h, 0, 0)),
            pl.BlockSpec((1, DH, D), lambda i, h: (h, 0, 0)),
            pl.BlockSpec((1, D), lambda i, h: (0, 0)),
            pl.BlockSpec((1, D), lambda i, h: (0, 0)),
            pl.BlockSpec((1, D), lambda i, h: (0, 0)),
        ],
        out_specs=[
            pl.BlockSpec((1, QB, S), lambda i, h: (h, i, 0)),
            pl.BlockSpec((QB, D), lambda i, h: (i, 0)),
        ],
        out_shape=[
            jax.ShapeDtypeStruct((H, S, S), jnp.float32),
            jax.ShapeDtypeStruct((S, D), jnp.float32),
        ],
        scratch_shapes=[
            pltpu.VMEM((S, D), jnp.bfloat16),
            pltpu.VMEM((H, S, DH), jnp.bfloat16),
            pltpu.VMEM((H, S, DH), jnp.bfloat16),
            pltpu.VMEM((QB, D), jnp.float32),
        ],
    )(x2d, Wqh, Wkh, Wvh, bqh, bkh, bvh, Woh,
      bo.reshape(1, D), g1.reshape(1, D), be1.reshape(1, D))

    b1r = b1.reshape(N_CHUNKS, 1, FC)
    out2d, aux = pl.pallas_call(
        _moe_kernel,
        grid=(N_CHUNKS,),
        in_specs=[
            pl.BlockSpec((S, D), lambda kk: (0, 0)),
            pl.BlockSpec((D, N_EXPERTS_), lambda kk: (0, 0)),
            pl.BlockSpec((1, D, FC),
                         lambda kk: (kk // CHUNKS_PER_EXPERT, 0, kk % CHUNKS_PER_EXPERT)),
            pl.BlockSpec((1, 1, FC), lambda kk: (kk, 0, 0)),
            pl.BlockSpec((1, FC, D),
                         lambda kk: (kk // CHUNKS_PER_EXPERT, kk % CHUNKS_PER_EXPERT, 0)),
            pl.BlockSpec((N_EXPERTS_, D), lambda kk: (0, 0)),
            pl.BlockSpec((1, D), lambda kk: (0, 0)),
            pl.BlockSpec((1, D), lambda kk: (0, 0)),
        ],
        out_specs=[
            pl.BlockSpec((S, D), lambda kk: (0, 0)),
            pl.BlockSpec((1, 1), lambda kk: (0, 0)),
        ],
        out_shape=[
            jax.ShapeDtypeStruct((S, D), jnp.float32),
            jax.ShapeDtypeStruct((1, 1), jnp.float32),
        ],
        scratch_shapes=[
            pltpu.VMEM((S, N_EXPERTS_), jnp.float32),
            pltpu.VMEM((S, D), jnp.float32),
        ],
    )(x1, Wg, W1, b1r, W2, b2, g2.reshape(1, D), be2.reshape(1, D))

    out = out2d.reshape(Bx, S, D)
    attn = attn.reshape(Bx, H, S, S)
    return out, attn, aux.reshape(())


# speedup vs baseline: 1.0270x; 1.0270x over previous
import functools

import jax
import jax.numpy as jnp
from jax.experimental import pallas as pl
from jax.experimental.pallas import tpu as pltpu

D_MODEL_ = 768
D_FF_ = 3072
N_EXPERTS_ = 8
N_HEADS_ = 12
S_ = 2048
DH_ = D_MODEL_ // N_HEADS_

TB = 256
QB = 1024
FC = 1536
CHUNKS_PER_EXPERT = D_FF_ // FC
N_CHUNKS = N_EXPERTS_ * CHUNKS_PER_EXPERT


def _attn_kernel(x_ref, wq_ref, wk_ref, wv_ref, bq_ref, bk_ref, bv_ref,
                 wo_ref, bo_ref, g_ref, b_ref,
                 attn_ref, x1_ref, xb_scr, k_scr, v_scr, t_scr, *, scale):
    qb = pl.program_id(0)
    hh = pl.program_id(1)

    @pl.when(jnp.logical_and(hh == 0, qb == 0))
    def _xcast():
        xb_scr[...] = x_ref[...].astype(jnp.bfloat16)

    @pl.when(qb == 0)
    def _kv():
        xb = xb_scr[...]
        k = jnp.dot(xb, wk_ref[0], preferred_element_type=jnp.float32) + bk_ref[0]
        v = jnp.dot(xb, wv_ref[0], preferred_element_type=jnp.float32) + bv_ref[0]
        k_scr[hh] = k.astype(jnp.bfloat16)
        v_scr[hh] = v.astype(jnp.bfloat16)

    @pl.when(hh == 0)
    def _resid():
        t_scr[...] = x_ref[pl.ds(qb * QB, QB), :] + bo_ref[...]

    xq = xb_scr[pl.ds(qb * QB, QB), :]
    q = jnp.dot(xq, wq_ref[0], preferred_element_type=jnp.float32) + bq_ref[0]
    s = jax.lax.dot_general(q.astype(jnp.bfloat16), k_scr[hh],
                            (((1,), (1,)), ((), ())),
                            preferred_element_type=jnp.float32) * scale
    m = jnp.max(s, axis=-1, keepdims=True)
    p = jnp.exp(s - m)
    a = p * (1.0 / jnp.sum(p, axis=-1, keepdims=True))
    attn_ref[...] = a[None]
    ctx = jnp.dot(a.astype(jnp.bfloat16), v_scr[hh],
                  preferred_element_type=jnp.float32)
    t_scr[...] += jnp.dot(ctx.astype(jnp.bfloat16), wo_ref[0],
                          preferred_element_type=jnp.float32)

    @pl.when(hh == N_HEADS_ - 1)
    def _ln():
        t = t_scr[...]
        mu = jnp.mean(t, axis=-1, keepdims=True)
        d = t - mu
        var = jnp.mean(d * d, axis=-1, keepdims=True)
        x1_ref[...] = d * jax.lax.rsqrt(var + 1e-5) * g_ref[...] + b_ref[...]


def _moe_kernel(x1_ref, wg_ref, w1_ref, b1_ref, w2_ref, b2_ref, g2_ref, be2_ref,
                out_ref, aux_ref, probs_scr, acc_scr):
    kk = pl.program_id(0)

    @pl.when(kk == 0)
    def _init():
        logits = jnp.dot(x1_ref[...], wg_ref[...], preferred_element_type=jnp.float32)
        lm = jnp.max(logits, axis=-1, keepdims=True)
        pe = jnp.exp(logits - lm)
        p = pe / jnp.sum(pe, axis=-1, keepdims=True)
        probs_scr[...] = p
        imp = jnp.mean(p, axis=0)
        aux_ref[...] = (N_EXPERTS_ * jnp.sum(imp * imp)).reshape(1, 1)
        acc_scr[...] = jnp.zeros_like(acc_scr)

    e = kk // CHUNKS_PER_EXPERT
    h = jnp.dot(x1_ref[...], w1_ref[0], preferred_element_type=jnp.float32) + b1_ref[0]
    h = jnp.maximum(h, 0.0)
    onehot = (jax.lax.broadcasted_iota(jnp.int32, (N_EXPERTS_, 1), 0) == e
              ).astype(jnp.float32)
    pcol = jnp.dot(probs_scr[...], onehot, preferred_element_type=jnp.float32)
    acc_scr[...] += pcol * jnp.dot(h, w2_ref[0], preferred_element_type=jnp.float32)

    @pl.when(kk == N_CHUNKS - 1)
    def _fin():
        pb2 = jnp.dot(probs_scr[...], b2_ref[...], preferred_element_type=jnp.float32)
        t = x1_ref[...] + acc_scr[...] + pb2
        mu = jnp.mean(t, axis=-1, keepdims=True)
        d = t - mu
        var = jnp.mean(d * d, axis=-1, keepdims=True)
        out_ref[...] = d * jax.lax.rsqrt(var + 1e-5) * g2_ref[...] + be2_ref[...]


def kernel(x, Wq, bq, Wk, bk, Wv, bv, Wo, bo, g1, be1, g2, be2, Wg, W1, b1, W2, b2):
    Bx, S, D = x.shape
    H, DH = N_HEADS_, DH_
    x2d = x.reshape(S, D)

    Wqh = Wq.reshape(D, H, DH).transpose(1, 0, 2).astype(jnp.bfloat16)
    Wkh = Wk.reshape(D, H, DH).transpose(1, 0, 2).astype(jnp.bfloat16)
    Wvh = Wv.reshape(D, H, DH).transpose(1, 0, 2).astype(jnp.bfloat16)
    Woh = Wo.reshape(H, DH, D).astype(jnp.bfloat16)
    bqh = bq.reshape(H, 1, DH)
    bkh = bk.reshape(H, 1, DH)
    bvh = bv.reshape(H, 1, DH)

    nq = S // QB
    attn, x1 = pl.pallas_call(
        functools.partial(_attn_kernel, scale=1.0 / (DH ** 0.5)),
        grid=(nq, H),
        in_specs=[
            pl.BlockSpec((S, D), lambda i, h: (0, 0)),
            pl.BlockSpec((1, D, DH), lambda i, h: (h, 0, 0)),
            pl.BlockSpec((1, D, DH), lambda i, h: (h, 0, 0)),
            pl.BlockSpec((1, D, DH), lambda i, h: (h, 0, 0)),
            pl.BlockSpec((1, 1, DH), lambda i, h: (h, 0, 0)),
            pl.BlockSpec((1, 1, DH), lambda i, h: (h, 0, 0)),
            pl.BlockSpec((1, 1, DH), lambda i, h: (h, 0, 0)),
            pl.BlockSpec((1, DH, D), lambda i, h: (h, 0, 0)),
            pl.BlockSpec((1, D), lambda i, h: (0, 0)),
            pl.BlockSpec((1, D), lambda i, h: (0, 0)),
            pl.BlockSpec((1, D), lambda i, h: (0, 0)),
        ],
        out_specs=[
            pl.BlockSpec((1, QB, S), lambda i, h: (h, i, 0)),
            pl.BlockSpec((QB, D), lambda i, h: (i, 0)),
        ],
        out_shape=[
            jax.ShapeDtypeStruct((H, S, S), jnp.float32),
            jax.ShapeDtypeStruct((S, D), jnp.float32),
        ],
        scratch_shapes=[
            pltpu.VMEM((S, D), jnp.bfloat16),
            pltpu.VMEM((H, S, DH), jnp.bfloat16),
            pltpu.VMEM((H, S, DH), jnp.bfloat16),
            pltpu.VMEM((QB, D), jnp.float32),
        ],
    )(x2d, Wqh, Wkh, Wvh, bqh, bkh, bvh, Woh,
      bo.reshape(1, D), g1.reshape(1, D), be1.reshape(1, D))

    b1r = b1.reshape(N_CHUNKS, 1, FC)
    out2d, aux = pl.pallas_call(
        _moe_kernel,
        grid=(N_CHUNKS,),
        in_specs=[
            pl.BlockSpec((S, D), lambda kk: (0, 0)),
            pl.BlockSpec((D, N_EXPERTS_), lambda kk: (0, 0)),
            pl.BlockSpec((1, D, FC),
                         lambda kk: (kk // CHUNKS_PER_EXPERT, 0, kk % CHUNKS_PER_EXPERT)),
            pl.BlockSpec((1, 1, FC), lambda kk: (kk, 0, 0)),
            pl.BlockSpec((1, FC, D),
                         lambda kk: (kk // CHUNKS_PER_EXPERT, kk % CHUNKS_PER_EXPERT, 0)),
            pl.BlockSpec((N_EXPERTS_, D), lambda kk: (0, 0)),
            pl.BlockSpec((1, D), lambda kk: (0, 0)),
            pl.BlockSpec((1, D), lambda kk: (0, 0)),
        ],
        out_specs=[
            pl.BlockSpec((S, D), lambda kk: (0, 0)),
            pl.BlockSpec((1, 1), lambda kk: (0, 0)),
        ],
        out_shape=[
            jax.ShapeDtypeStruct((S, D), jnp.float32),
            jax.ShapeDtypeStruct((1, 1), jnp.float32),
        ],
        scratch_shapes=[
            pltpu.VMEM((S, N_EXPERTS_), jnp.float32),
            pltpu.VMEM((S, D), jnp.float32),
        ],
    )(x1, Wg, W1, b1r, W2, b2, g2.reshape(1, D), be2.reshape(1, D))

    out = out2d.reshape(Bx, S, D)
    attn = attn.reshape(Bx, H, S, S)
    return out, attn, aux.reshape(())
